# baseline (speedup 1.0000x reference)
import jax
import jax.numpy as jnp
from jax.experimental import pallas as pl
from jax.experimental.pallas import tpu as pltpu

_UPER = 100
_ALPHA = 1.0
_GAMMA = 0.5
_MIN_KEEP = 1


def _fused_kernel(x_ref, tgt_ref, out_ref, v_s, acc_s):
    i = pl.program_id(0)
    nb = pl.num_programs(0)
    x = x_ref[...]
    tgt = tgt_ref[...]
    R, C = x.shape
    tgt_col = tgt[:, None]
    m = jnp.max(x, axis=1, keepdims=True)
    e = jnp.exp(x - m)
    s = jax.lax.dot_general(e, jnp.ones((C, 1), jnp.float32),
                            (((1,), (0,)), ((), ())),
                            preferred_element_type=jnp.float32)
    col = jax.lax.broadcasted_iota(jnp.int32, (R, C), 1)
    colf = col.astype(jnp.float32)
    predrev = jnp.max(jnp.where(x == m, C - colf, 0.0), axis=1, keepdims=True)
    xt = jnp.max(jnp.where(col == tgt_col, x, -jnp.inf), axis=1, keepdims=True)
    l = m + jnp.log(s) - xt
    pred_f = C - predrev
    w = _ALPHA * jnp.sqrt(jnp.abs(pred_f - tgt_col.astype(jnp.float32)))
    num_mask = tgt_col < _UPER
    vblk = jnp.where(num_mask, -1.0, l)
    v_s[pl.ds(i * R, R)] = vblk[:, 0]
    sum_l = jnp.sum(l)
    numer = jnp.sum(jnp.where(num_mask, (1.0 + w) * l, 0.0))
    n_num_p = jnp.sum(num_mask.astype(jnp.float32))
    lane = jax.lax.broadcasted_iota(jnp.int32, (1, 128), 1)
    part = (jnp.where(lane == 0, sum_l, 0.0)
            + jnp.where(lane == 1, numer, 0.0)
            + jnp.where(lane == 2, n_num_p, 0.0))

    @pl.when(i == 0)
    def _():
        acc_s[...] = jnp.zeros_like(acc_s)

    acc_s[...] += part

    @pl.when(i == nb - 1)
    def _():
        v = v_s[...]
        bf = jnp.float32(v.shape[0])
        sum_l_t = acc_s[0, 0]
        numer_sum = acc_s[0, 1]
        n_num = acc_s[0, 2]
        celoss = sum_l_t / bf
        n_cls_i = jnp.int32(v.shape[0]) - n_num.astype(jnp.int32)
        cls_sum_all = jnp.sum(jnp.where(v >= 0.0, v, 0.0))
        k = (7 * n_cls_i) // 10

        def body(_, lohi):
            lo, hi = lohi
            mid = lo + (hi - lo + 1) // 2
            t = jax.lax.bitcast_convert_type(mid, jnp.float32)
            ge = jnp.sum((v >= t).astype(jnp.int32)) >= k
            return (jnp.where(ge, mid, lo), jnp.where(ge, hi, mid - 1))

        lo, _ = jax.lax.fori_loop(0, 32, body,
                                  (jnp.int32(0), jnp.int32(0x7F800000)))
        t = jax.lax.bitcast_convert_type(lo, jnp.float32)
        gt = v > t
        cnt_gt = jnp.sum(gt.astype(jnp.int32))
        sum_gt = jnp.sum(jnp.where(gt, v, 0.0))
        topk_sum = sum_gt + (k - cnt_gt).astype(jnp.float32) * t

        use_topk = k >= _MIN_KEEP
        cls_sum = jnp.where(n_cls_i > 0,
                            jnp.where(use_topk, topk_sum, cls_sum_all), 0.0)
        valid_num = jnp.where(use_topk, k.astype(jnp.float32), float(_MIN_KEEP))
        cls_size = jnp.where(n_cls_i > 0, valid_num, 0.0)
        numerical_loss = (cls_sum + numer_sum) / (n_num + cls_size + 1e-9)
        out_ref[0] = numerical_loss
        out_ref[1] = celoss / numerical_loss
        out_ref[2] = cls_sum / (cls_size + 1e-9)
        out_ref[3] = numer_sum / (n_num + 1e-9)
        out_ref[4] = cls_size
        out_ref[5] = n_cls_i.astype(jnp.float32)
        out_ref[6] = n_num


def kernel(inputs, targets):
    B, C = inputs.shape
    R = 2048
    out = pl.pallas_call(
        _fused_kernel,
        grid=(B // R,),
        in_specs=[
            pl.BlockSpec((R, C), lambda i: (i, 0)),
            pl.BlockSpec((R,), lambda i: (i,)),
        ],
        out_specs=pl.BlockSpec(memory_space=pltpu.SMEM),
        out_shape=jax.ShapeDtypeStruct((8,), jnp.float32),
        scratch_shapes=[
            pltpu.VMEM((B,), jnp.float32),
            pltpu.VMEM((1, 128), jnp.float32),
        ],
    )(inputs, targets)

    return (out[0], out[1], out[2], out[3], out[4], out[5], out[6])


# speedup vs baseline: 1.0096x; 1.0096x over previous
import jax
import jax.numpy as jnp
from jax.experimental import pallas as pl
from jax.experimental.pallas import tpu as pltpu

_UPER = 100
_ALPHA = 1.0
_GAMMA = 0.5
_MIN_KEEP = 1


def _fused_kernel(x_ref, tgt_ref, out_ref, v_s, acc_s):
    i = pl.program_id(0)
    nb = pl.num_programs(0)
    x = x_ref[...]
    tgt = tgt_ref[...]
    R, C = x.shape
    tgt_col = tgt[:, None]
    m = jnp.max(x, axis=1, keepdims=True)
    e = jnp.exp(x - m)
    s = jax.lax.dot_general(e, jnp.ones((C, 1), jnp.float32),
                            (((1,), (0,)), ((), ())),
                            preferred_element_type=jnp.float32)
    col = jax.lax.broadcasted_iota(jnp.int32, (R, C), 1)
    colf = col.astype(jnp.float32)
    predrev = jnp.max(jnp.where(x == m, C - colf, 0.0), axis=1, keepdims=True)
    xt = jnp.max(jnp.where(col == tgt_col, x, -jnp.inf), axis=1, keepdims=True)
    l = m + jnp.log(s) - xt
    pred_f = C - predrev
    w = _ALPHA * jnp.sqrt(jnp.abs(pred_f - tgt_col.astype(jnp.float32)))
    num_mask = tgt_col < _UPER
    vblk = jnp.where(num_mask, -1.0, l)
    v_s[pl.ds(i * R, R)] = vblk[:, 0]
    sum_l = jnp.sum(l)
    numer = jnp.sum(jnp.where(num_mask, (1.0 + w) * l, 0.0))
    n_num_p = jnp.sum(num_mask.astype(jnp.float32))
    lane = jax.lax.broadcasted_iota(jnp.int32, (1, 128), 1)
    part = (jnp.where(lane == 0, sum_l, 0.0)
            + jnp.where(lane == 1, numer, 0.0)
            + jnp.where(lane == 2, n_num_p, 0.0))

    @pl.when(i == 0)
    def _():
        acc_s[...] = jnp.zeros_like(acc_s)

    acc_s[...] += part

    @pl.when(i == nb - 1)
    def _():
        v = v_s[...]
        bf = jnp.float32(v.shape[0])
        sum_l_t = acc_s[0, 0]
        numer_sum = acc_s[0, 1]
        n_num = acc_s[0, 2]
        celoss = sum_l_t / bf
        n_cls_i = jnp.int32(v.shape[0]) - n_num.astype(jnp.int32)
        cls_sum_all = jnp.sum(jnp.where(v >= 0.0, v, 0.0))
        k = (7 * n_cls_i) // 10

        def body(_, lohi):
            lo, hi = lohi
            mid = lo + (hi - lo + 1) // 2
            t = jax.lax.bitcast_convert_type(mid, jnp.float32)
            ge = jnp.sum((v >= t).astype(jnp.int32)) >= k
            return (jnp.where(ge, mid, lo), jnp.where(ge, hi, mid - 1))

        lo, _ = jax.lax.fori_loop(0, 32, body,
                                  (jnp.int32(0), jnp.int32(0x7F800000)))
        t = jax.lax.bitcast_convert_type(lo, jnp.float32)
        gt = v > t
        cnt_gt = jnp.sum(gt.astype(jnp.int32))
        sum_gt = jnp.sum(jnp.where(gt, v, 0.0))
        topk_sum = sum_gt + (k - cnt_gt).astype(jnp.float32) * t

        use_topk = k >= _MIN_KEEP
        cls_sum = jnp.where(n_cls_i > 0,
                            jnp.where(use_topk, topk_sum, cls_sum_all), 0.0)
        valid_num = jnp.where(use_topk, k.astype(jnp.float32), float(_MIN_KEEP))
        cls_size = jnp.where(n_cls_i > 0, valid_num, 0.0)
        numerical_loss = (cls_sum + numer_sum) / (n_num + cls_size + 1e-9)
        out_ref[0] = numerical_loss
        out_ref[1] = celoss / numerical_loss
        out_ref[2] = cls_sum / (cls_size + 1e-9)
        out_ref[3] = numer_sum / (n_num + 1e-9)
        out_ref[4] = cls_size
        out_ref[5] = n_cls_i.astype(jnp.float32)
        out_ref[6] = n_num


def kernel(inputs, targets):
    B, C = inputs.shape
    R = 1024
    out = pl.pallas_call(
        _fused_kernel,
        grid=(B // R,),
        in_specs=[
            pl.BlockSpec((R, C), lambda i: (i, 0)),
            pl.BlockSpec((R,), lambda i: (i,)),
        ],
        out_specs=pl.BlockSpec(memory_space=pltpu.SMEM),
        out_shape=jax.ShapeDtypeStruct((8,), jnp.float32),
        scratch_shapes=[
            pltpu.VMEM((B,), jnp.float32),
            pltpu.VMEM((1, 128), jnp.float32),
        ],
    )(inputs, targets)

    return (out[0], out[1], out[2], out[3], out[4], out[5], out[6])
